# baseline (speedup 1.0000x reference)
import functools

import jax
import jax.numpy as jnp
from jax import lax
from jax.experimental import pallas as pl
from jax.experimental.pallas import tpu as pltpu
from jax.experimental.pallas import tpu_sc as plsc

E = 8
TOPK = 2
H = 768
FF = 768
M = 2048
P = M * TOPK
TM = 256
NT = P // TM
NWK = 24
NW = 32
RPW = P // NW


def _shift_down(a, k):
    return jnp.concatenate(
        [jnp.zeros((k, a.shape[1]), a.dtype), a[:-k, :]], axis=0)


def _shift_right(a, k):
    return jnp.concatenate(
        [jnp.zeros((a.shape[0], k), a.dtype), a[:, :-k]], axis=1)


def _router_body(x_ref, gw_ref, logits_ref, wflat_ref, dst_ref,
                 offs_ref, wle_ref, wlt_ref):
    x = x_ref[...]
    logits = lax.dot_general(x, gw_ref[...], (((1,), (1,)), ((), ())),
                             preferred_element_type=jnp.float32)
    logits_ref[...] = logits

    lane = lax.broadcasted_iota(jnp.int32, (M, E), 1)
    m1 = jnp.max(logits, axis=1, keepdims=True)
    a1 = jnp.min(jnp.where(logits == m1, lane, E), axis=1, keepdims=True)
    l2m = jnp.where(lane == a1, -jnp.inf, logits)
    m2 = jnp.max(l2m, axis=1, keepdims=True)
    a2 = jnp.min(jnp.where(l2m == m2, lane, E), axis=1, keepdims=True)
    w1 = 1.0 / (1.0 + jnp.exp(m2 - m1))
    wflat_ref[...] = jnp.concatenate([w1, 1.0 - w1], axis=1)

    oh1 = (lane == a1).astype(jnp.int32)
    oh2 = (lane == a2).astype(jnp.int32)
    c = oh1 + oh2
    incl = c
    for k in (1, 2, 4, 8, 16, 32, 64, 128, 256, 512, 1024):
        incl = incl + _shift_down(incl, k)
    excl = incl - c

    tot_row = jnp.sum(c, axis=0, keepdims=True)
    acc = _shift_right(tot_row, 1)
    acc = acc + _shift_right(acc, 1)
    acc = acc + _shift_right(acc, 2)
    acc = acc + _shift_right(acc, 4)
    offs_row = acc

    rank1 = jnp.sum(oh1 * excl, axis=1, keepdims=True)
    rank2 = jnp.sum(oh2 * (excl + oh1), axis=1, keepdims=True)
    offs_bcast = jnp.broadcast_to(offs_row, (M, E))
    off_a1 = jnp.sum(oh1 * offs_bcast, axis=1, keepdims=True)
    off_a2 = jnp.sum(oh2 * offs_bcast, axis=1, keepdims=True)
    dst_ref[...] = jnp.concatenate([off_a1 + rank1, off_a2 + rank2], axis=1)

    offs16 = jnp.concatenate(
        [offs_row, jnp.full((1, 1), P, jnp.int32),
         jnp.zeros((1, 7), jnp.int32)], axis=1)
    offs_ref[...] = jnp.broadcast_to(offs16, (8, 16))

    ones_col = jnp.ones((M, 1), jnp.float32)
    tot_col = lax.dot_general(c.astype(jnp.float32), ones_col,
                              (((0,), (0,)), ((), ())),
                              precision=lax.Precision.HIGHEST,
                              preferred_element_type=jnp.float32)
    tot_col = tot_col.astype(jnp.int32)
    inc = tot_col
    inc = inc + _shift_down(inc, 1)
    inc = inc + _shift_down(inc, 2)
    inc = inc + _shift_down(inc, 4)
    offs_col = inc - tot_col
    hi_col = offs_col + tot_col
    lo_t = offs_col >> 8
    hi_t = (hi_col + (TM - 1)) >> 8
    count = jnp.where(tot_col > 0, hi_t - lo_t, 0)
    sinc = count
    sinc = sinc + _shift_down(sinc, 1)
    sinc = sinc + _shift_down(sinc, 2)
    sinc = sinc + _shift_down(sinc, 4)
    start = sinc - count

    p_iota = lax.broadcasted_iota(jnp.int32, (E, 32), 1)
    b = (start <= p_iota).astype(jnp.int32)
    bsel = b - jnp.concatenate([b[1:, :], jnp.zeros((1, 32), jnp.int32)],
                               axis=0)
    wle = jnp.sum(b, axis=0, keepdims=True) - 1
    start_at = jnp.sum(bsel * start, axis=0, keepdims=True)
    lo_at = jnp.sum(bsel * lo_t, axis=0, keepdims=True)
    hi_at = jnp.sum(bsel * hi_t, axis=0, keepdims=True)
    p_row = lax.broadcasted_iota(jnp.int32, (1, 32), 1)
    wlt = lo_at + (p_row - start_at)
    wlt = jnp.clip(jnp.minimum(wlt, hi_at - 1), 0, NT - 1)
    wle_ref[...] = jnp.broadcast_to(wle, (8, 32))
    wlt_ref[...] = jnp.broadcast_to(wlt, (8, 32))


def _router(x, gate_w):
    return pl.pallas_call(
        _router_body,
        out_shape=(
            jax.ShapeDtypeStruct((M, E), jnp.float32),
            jax.ShapeDtypeStruct((M, TOPK), jnp.float32),
            jax.ShapeDtypeStruct((M, TOPK), jnp.int32),
            jax.ShapeDtypeStruct((8, 16), jnp.int32),
            jax.ShapeDtypeStruct((8, 32), jnp.int32),
            jax.ShapeDtypeStruct((8, 32), jnp.int32),
        ),
    )(x, gate_w)


def _sc_permute(table, idx, gather_by_idx):
    rows, width = (P, table.shape[1])
    mesh = plsc.VectorSubcoreMesh(core_axis_name="c", subcore_axis_name="s")

    @functools.partial(
        pl.kernel,
        out_type=jax.ShapeDtypeStruct((rows, width), table.dtype),
        mesh=mesh,
        scratch_types=[
            pltpu.VMEM((RPW,), jnp.int32),
            pltpu.VMEM((RPW,), jnp.int32),
            pltpu.VMEM((RPW, width), table.dtype),
            pltpu.SemaphoreType.DMA,
        ],
    )
    def k(table_hbm, idx_hbm, out_hbm, idx_v, tok_v, rows_v, sem):
        wid = lax.axis_index("s") * 2 + lax.axis_index("c")
        base = wid * RPW
        pltpu.sync_copy(idx_hbm.at[pl.ds(base, RPW)], idx_v)
        if gather_by_idx:
            pltpu.async_copy(table_hbm.at[idx_v], rows_v, sem).wait()
            pltpu.sync_copy(rows_v, out_hbm.at[pl.ds(base, RPW)])
        else:
            @pl.loop(0, RPW // 16)
            def _(cc):
                tok_v[pl.ds(cc * 16, 16)] = lax.shift_right_logical(
                    base + cc * 16 + lax.iota(jnp.int32, 16), 1)
            pltpu.async_copy(table_hbm.at[tok_v], rows_v, sem).wait()
            pltpu.async_copy(rows_v, out_hbm.at[idx_v], sem).wait()

    return k(table, idx)


def _gemm_body(offs_ref, wle_ref, wlt_ref, xs_ref, wg_ref, wu_ref, wd_ref,
               ys_ref):
    p = pl.program_id(0)
    e = wle_ref[p]
    t = wlt_ref[p]
    lo = offs_ref[e]
    hi = offs_ref[e + 1]
    row0 = t * TM

    @pl.when((hi > row0) & (lo < row0 + TM))
    def _():
        sl = pl.ds(row0, TM)
        xt = xs_ref[sl, :].astype(jnp.bfloat16)
        g = lax.dot_general(xt, wg_ref[0], (((1,), (1,)), ((), ())),
                            preferred_element_type=jnp.float32)
        u = lax.dot_general(xt, wu_ref[0], (((1,), (1,)), ((), ())),
                            preferred_element_type=jnp.float32)
        h = (g / (1.0 + jnp.exp(-g))) * u
        y = lax.dot_general(h.astype(jnp.bfloat16), wd_ref[0],
                            (((1,), (1,)), ((), ())),
                            preferred_element_type=jnp.float32)
        rows = row0 + lax.broadcasted_iota(jnp.int32, (TM, 1), 0)
        mask = (rows >= lo) & (rows < hi)
        ys_ref[sl, :] = jnp.where(mask, y, ys_ref[sl, :])


def _grouped_gemm(offs16, wle, wlt, xs, wg_bf, wu_bf, wd_bf):
    grid_spec = pltpu.PrefetchScalarGridSpec(
        num_scalar_prefetch=3,
        grid=(NWK,),
        in_specs=[
            pl.BlockSpec((P, H), lambda p, o, we, wt: (0, 0)),
            pl.BlockSpec((1, FF, H), lambda p, o, we, wt: (we[p], 0, 0)),
            pl.BlockSpec((1, FF, H), lambda p, o, we, wt: (we[p], 0, 0)),
            pl.BlockSpec((1, H, FF), lambda p, o, we, wt: (we[p], 0, 0)),
        ],
        out_specs=pl.BlockSpec((P, H), lambda p, o, we, wt: (0, 0)),
    )
    return pl.pallas_call(
        _gemm_body,
        grid_spec=grid_spec,
        out_shape=jax.ShapeDtypeStruct((P, H), jnp.float32),
        compiler_params=pltpu.CompilerParams(
            dimension_semantics=("arbitrary",)),
    )(offs16, wle, wlt, xs, wg_bf, wu_bf, wd_bf)


def _combine_body(hu_ref, wflat_ref, out_ref):
    h = hu_ref[...]
    w1 = wflat_ref[:, 0:1]
    w2 = wflat_ref[:, 1:2]
    out_ref[...] = (h[:, :H].astype(jnp.float32) * w1
                    + h[:, H:].astype(jnp.float32) * w2)


def _combine(hu2, wflat):
    return pl.pallas_call(
        _combine_body,
        out_shape=jax.ShapeDtypeStruct((M, H), jnp.float32),
    )(hu2, wflat)


@jax.jit
def kernel(hidden_states, gate_w, gate_proj_w, up_proj_w, down_proj_w):
    B_, S_, H_ = hidden_states.shape
    x = hidden_states.reshape(M, H)

    (logits, wflat, dst2, offs_o, wle_o, wlt_o) = _router(x, gate_w)
    dst = dst2.reshape(P)
    offs16 = offs_o[0]
    wle = wle_o[0]
    wlt = wlt_o[0]

    wg_bf = gate_proj_w.astype(jnp.bfloat16)
    wu_bf = up_proj_w.astype(jnp.bfloat16)
    wd_bf = down_proj_w.astype(jnp.bfloat16)

    xs = _sc_permute(x, dst, gather_by_idx=False)
    ys = _grouped_gemm(offs16, wle, wlt, xs, wg_bf, wu_bf, wd_bf)
    hu = _sc_permute(ys, dst, gather_by_idx=True)
    out = _combine(hu.reshape(M, TOPK * H), wflat)

    return out.reshape(B_, S_, H_), logits
